# baseline (speedup 1.0000x reference)
import math

import jax
import jax.numpy as jnp
import numpy as np
from jax.experimental import pallas as pl
from jax.experimental.pallas import tpu as pltpu

DIM = 1024
SCALE = math.sqrt(DIM)
LO = 512
K = 8
LSUB = LO // K
MAX_SEQ = 4096
_FEAT = 4


def _make_tables(max_seq, feat):
    d = np.arange(DIM)
    freq = np.exp(-(2 * (d // 2)).astype(np.float64) * (math.log(10000.0) / DIM))
    even = (d % 2) == 0

    n_hi = max_seq // LSUB
    hi_angle = (LSUB * np.arange(n_hi, dtype=np.float64))[:, None] * freq[None, :]
    p_hi = np.where(even[None, :], np.sin(hi_angle), np.cos(hi_angle))
    q_hi = np.where(even[None, :], np.cos(hi_angle), -np.sin(hi_angle))

    lo_angle = np.arange(LSUB, dtype=np.float64)[:, None] * freq[None, :]
    c_lo = np.repeat(np.cos(lo_angle), feat, axis=0)
    s_lo = np.repeat(np.sin(lo_angle), feat, axis=0)

    return (
        np.repeat(p_hi.astype(np.float32)[:, None, :], 8, axis=1),
        np.repeat(q_hi.astype(np.float32)[:, None, :], 8, axis=1),
        c_lo.astype(np.float32),
        s_lo.astype(np.float32),
    )


_TABLES = _make_tables(MAX_SEQ, _FEAT)
_LROWS = LSUB * _FEAT


def _block_kernel(emb_ref, p_ref, q_ref, cl_ref, sl_ref, out_ref):
    g = _LROWS // 8
    cl = cl_ref[...].reshape(1, g, 8, DIM)
    sl = sl_ref[...].reshape(1, g, 8, DIM)
    p = p_ref[...][:, None]
    q = q_ref[...][:, None]
    e = emb_ref[...].reshape(K, g, 8, DIM)
    out = e * SCALE + (cl * p + sl * q)
    out_ref[...] = out.reshape(LO, _FEAT, DIM)


def kernel(emb):
    seq, feat, dim = emb.shape
    return pl.pallas_call(
        _block_kernel,
        grid=(seq // LO,),
        in_specs=[
            pl.BlockSpec((LO, feat, dim), lambda i: (i, 0, 0)),
            pl.BlockSpec((K, 8, dim), lambda i: (i, 0, 0)),
            pl.BlockSpec((K, 8, dim), lambda i: (i, 0, 0)),
            pl.BlockSpec((_LROWS, dim), lambda i: (0, 0)),
            pl.BlockSpec((_LROWS, dim), lambda i: (0, 0)),
        ],
        out_specs=pl.BlockSpec((LO, feat, dim), lambda i: (i, 0, 0)),
        out_shape=jax.ShapeDtypeStruct((seq, feat, dim), emb.dtype),
        compiler_params=pltpu.CompilerParams(dimension_semantics=("parallel",)),
    )(emb, *_TABLES)


# speedup vs baseline: 1.0042x; 1.0042x over previous
import math

import jax
import jax.numpy as jnp
import numpy as np
from jax.experimental import pallas as pl

DIM = 1024
SCALE = math.sqrt(DIM)
LO = 512
K = 8
LSUB = LO // K
MAX_SEQ = 4096
_FEAT = 4


def _make_tables(max_seq, feat):
    d = np.arange(DIM)
    freq = np.exp(-(2 * (d // 2)).astype(np.float64) * (math.log(10000.0) / DIM))
    even = (d % 2) == 0

    n_hi = max_seq // LSUB
    hi_angle = (LSUB * np.arange(n_hi, dtype=np.float64))[:, None] * freq[None, :]
    p_hi = np.where(even[None, :], np.sin(hi_angle), np.cos(hi_angle))
    q_hi = np.where(even[None, :], np.cos(hi_angle), -np.sin(hi_angle))

    lo_angle = np.arange(LSUB, dtype=np.float64)[:, None] * freq[None, :]
    c_lo = np.repeat(np.cos(lo_angle), feat, axis=0)
    s_lo = np.repeat(np.sin(lo_angle), feat, axis=0)

    return (
        np.repeat(p_hi.astype(np.float32)[:, None, :], 8, axis=1),
        np.repeat(q_hi.astype(np.float32)[:, None, :], 8, axis=1),
        c_lo.astype(np.float32),
        s_lo.astype(np.float32),
    )


_TABLES = _make_tables(MAX_SEQ, _FEAT)
_LROWS = LSUB * _FEAT


def _block_kernel(emb_ref, p_ref, q_ref, cl_ref, sl_ref, out_ref):
    g = _LROWS // 8
    cl = cl_ref[...].reshape(1, g, 8, DIM)
    sl = sl_ref[...].reshape(1, g, 8, DIM)
    p = p_ref[...][:, None]
    q = q_ref[...][:, None]
    e = emb_ref[...].reshape(K, g, 8, DIM)
    out = e * SCALE + (cl * p + sl * q)
    out_ref[...] = out.reshape(LO, _FEAT, DIM)


def kernel(emb):
    seq, feat, dim = emb.shape
    return pl.pallas_call(
        _block_kernel,
        grid=(seq // LO,),
        in_specs=[
            pl.BlockSpec((LO, feat, dim), lambda i: (i, 0, 0)),
            pl.BlockSpec((K, 8, dim), lambda i: (i, 0, 0)),
            pl.BlockSpec((K, 8, dim), lambda i: (i, 0, 0)),
            pl.BlockSpec((_LROWS, dim), lambda i: (0, 0)),
            pl.BlockSpec((_LROWS, dim), lambda i: (0, 0)),
        ],
        out_specs=pl.BlockSpec((LO, feat, dim), lambda i: (i, 0, 0)),
        out_shape=jax.ShapeDtypeStruct((seq, feat, dim), emb.dtype),
    )(emb, *_TABLES)
